# Initial kernel scaffold; baseline (speedup 1.0000x reference)
import jax
import jax.numpy as jnp
from jax.experimental import pallas as pl


def kernel(x, edge_index, W1, b1, W2, b2):
    raise NotImplementedError("write your pallas kernel here")


# speedup vs baseline: 28.4455x; 28.4455x over previous
import functools

import jax
import jax.numpy as jnp
from jax import lax
from jax.experimental import pallas as pl
from jax.experimental.pallas import tpu as pltpu
from jax.experimental.pallas import tpu_sc as plsc

N = 10000
E = 320000
D = 128
C = 40

NSC = 2
NTILE = 16

B = 128
CH = 160
EPT = CH * B
E_PAD = NTILE * EPT

N_AGG = 10112
SLAB = N_AGG // NTILE

H1 = D // 2
W2P = 64
H2 = W2P // 2

RB = 1000
GRID = N // RB

_mesh = plsc.VectorSubcoreMesh(core_axis_name="c", subcore_axis_name="s")


def _make_agg(W):
  @functools.partial(
      pl.kernel,
      mesh=_mesh,
      compiler_params=pltpu.CompilerParams(use_tc_tiling_on_sc=False),
      out_type=jax.ShapeDtypeStruct((NSC, N_AGG, W), jnp.float32),
      scratch_types=[
          pltpu.VMEM((4, 2, B), jnp.int32),
          pltpu.VMEM((2, B, W), jnp.float32),
          pltpu.VMEM_SHARED((N_AGG, W), jnp.float32),
          pltpu.SemaphoreType.DMA,
          pltpu.SemaphoreType.DMA,
          pltpu.SemaphoreType.DMA,
          pltpu.SemaphoreType.DMA,
          pltpu.SemaphoreType.DMA,
          pltpu.SemaphoreType.DMA,
      ],
  )
  def agg(hs, idxi, zeros_w, out, iring, rows, agg_sh, i0, i1, i2, i3, g0, g1):
    cid = lax.axis_index("c")
    sid = lax.axis_index("s")
    isems = (i0, i1, i2, i3)
    gsems = (g0, g1)
    hsrc = hs.at[cid]

    pltpu.sync_copy(
        zeros_w.at[pl.ds(sid * SLAB, SLAB)],
        agg_sh.at[pl.ds(sid * SLAB, SLAB)],
    )
    plsc.subcore_barrier()

    for j in range(4):
      pltpu.async_copy(idxi.at[sid, j], iring.at[j], isems[j])
    pltpu.make_async_copy(idxi.at[sid, 0], iring.at[0], isems[0]).wait()
    pltpu.async_copy(hsrc.at[iring.at[0, 0]], rows.at[0], gsems[0])

    def body(c0):
      for j in range(4):
        ch = c0 + j
        sj = j
        rj = j % 2
        nrj = (j + 1) % 2
        nsj = (j + 1) % 4

        @pl.when(ch + 1 < CH)
        def _():
          pltpu.make_async_copy(
              idxi.at[sid, ch + 1], iring.at[nsj], isems[nsj]
          ).wait()
          pltpu.async_copy(
              hsrc.at[iring.at[nsj, 0]], rows.at[nrj], gsems[nrj]
          )

        pltpu.make_async_copy(
            hsrc.at[iring.at[sj, 0]], rows.at[rj], gsems[rj]
        ).wait()
        pltpu.sync_copy(rows.at[rj], agg_sh.at[iring.at[sj, 1]], add=True)

        @pl.when(ch + 4 < CH)
        def _():
          pltpu.async_copy(idxi.at[sid, ch + 4], iring.at[sj], isems[sj])

    pl.loop(0, CH, step=4)(body)

    plsc.subcore_barrier()
    pltpu.sync_copy(
        agg_sh.at[pl.ds(sid * SLAB, SLAB)],
        out.at[cid].at[pl.ds(sid * SLAB, SLAB)],
    )

  return agg


_agg1 = _make_agg(H1)
_agg2 = _make_agg(H2)


@functools.partial(
    pl.kernel,
    mesh=_mesh,
    compiler_params=pltpu.CompilerParams(use_tc_tiling_on_sc=False),
    out_type=jax.ShapeDtypeStruct((NSC, N_AGG, 16), jnp.float32),
    scratch_types=[
        pltpu.VMEM((4, 2, B), jnp.int32),
        pltpu.VMEM((B, 16), jnp.float32),
        pltpu.VMEM_SHARED((N_AGG, 16), jnp.float32),
        pltpu.SemaphoreType.DMA,
        pltpu.SemaphoreType.DMA,
        pltpu.SemaphoreType.DMA,
        pltpu.SemaphoreType.DMA,
    ],
)
def _deg(idxi, zeros16, out, iring, ones, deg_sh, i0, i1, i2, i3):
  cid = lax.axis_index("c")
  sid = lax.axis_index("s")
  isems = (i0, i1, i2, i3)
  base = cid * (CH // 2)

  for r in range(B):
    ones[r, :] = jnp.full((16,), 1.0, jnp.float32)

  pltpu.sync_copy(
      zeros16.at[pl.ds(sid * SLAB, SLAB)],
      deg_sh.at[pl.ds(sid * SLAB, SLAB)],
  )
  plsc.subcore_barrier()

  for j in range(4):
    pltpu.async_copy(idxi.at[sid, base + j], iring.at[j], isems[j])

  def body(c0):
    for j in range(4):
      ch = c0 + j
      sj = j
      pltpu.make_async_copy(
          idxi.at[sid, base + ch], iring.at[sj], isems[sj]
      ).wait()
      pltpu.sync_copy(ones, deg_sh.at[iring.at[sj, 1]], add=True)

      @pl.when(ch + 4 < CH // 2)
      def _():
        pltpu.async_copy(idxi.at[sid, base + ch + 4], iring.at[sj], isems[sj])

  pl.loop(0, CH // 2, step=4)(body)

  plsc.subcore_barrier()
  pltpu.sync_copy(
      deg_sh.at[pl.ds(sid * SLAB, SLAB)],
      out.at[cid].at[pl.ds(sid * SLAB, SLAB)],
  )


def _dinv_from(deg_ref):
  degsum = deg_ref[0, :, 0:1] + deg_ref[1, :, 0:1]
  return lax.rsqrt(degsum + 1.0)


def _mm1_body(x_ref, w_ref, deg_ref, o_ref):
  dinv = _dinv_from(deg_ref)
  h = dinv * jnp.dot(x_ref[...], w_ref[...], preferred_element_type=jnp.float32)
  o_ref[0, :, :] = h[:, :H1]
  o_ref[1, :, :] = h[:, H1:]


def _comb1_body(p_ref, hs1_ref, deg_ref, b1_ref, w2_ref, o_ref):
  dinv = _dinv_from(deg_ref)
  aggf = jnp.concatenate([p_ref[0], p_ref[1]], axis=1)
  hs1f = jnp.concatenate([hs1_ref[0], hs1_ref[1]], axis=1)
  h1 = jnp.maximum(dinv * (aggf + hs1f) + b1_ref[...], 0.0)
  s = dinv * jnp.dot(h1, w2_ref[...], preferred_element_type=jnp.float32)
  o_ref[0, :, :] = s[:, :H2]
  o_ref[1, :, :] = s[:, H2:]


def _final_body(q_ref, hs2_ref, deg_ref, b2_ref, o_ref):
  dinv = _dinv_from(deg_ref)
  qf = jnp.concatenate([q_ref[0], q_ref[1]], axis=1)
  hs2f = jnp.concatenate([hs2_ref[0], hs2_ref[1]], axis=1)
  z = dinv * (qf + hs2f)
  z40 = z[:, :C] + b2_ref[...]
  m = jnp.max(z40, axis=1, keepdims=True)
  e = z40 - m
  o_ref[...] = e - jnp.log(jnp.sum(jnp.exp(e), axis=1, keepdims=True))


def _deg_spec():
  return pl.BlockSpec((2, RB, 16), lambda i: (0, i, 0))


def kernel(x, edge_index, W1, b1, W2, b2):
  row = edge_index[0]
  col = edge_index[1]

  npad = E_PAD - E
  pad_pos = jnp.arange(npad, dtype=jnp.int32)
  row_pad = pad_pos % jnp.int32(9973)
  col_pad = jnp.int32(N) + pad_pos % jnp.int32(N_AGG - N)
  rowf = jnp.concatenate([row, row_pad]).reshape(NTILE, CH, B)
  colf = jnp.concatenate([col, col_pad]).reshape(NTILE, CH, B)
  idxi = jnp.stack([rowf, colf], axis=2)

  zeros16 = jnp.zeros((N_AGG, 16), jnp.float32)
  zeros_h1 = jnp.zeros((N_AGG, H1), jnp.float32)
  zeros_h2 = jnp.zeros((N_AGG, H2), jnp.float32)
  W2p = jnp.pad(W2, ((0, 0), (0, W2P - C)))

  degp = _deg(idxi, zeros16)

  hs1 = pl.pallas_call(
      _mm1_body,
      grid=(GRID,),
      in_specs=[
          pl.BlockSpec((RB, D), lambda i: (i, 0)),
          pl.BlockSpec((D, D), lambda i: (0, 0)),
          _deg_spec(),
      ],
      out_specs=pl.BlockSpec((2, RB, H1), lambda i: (0, i, 0)),
      out_shape=jax.ShapeDtypeStruct((NSC, N, H1), jnp.float32),
  )(x, W1, degp)

  p = _agg1(hs1, idxi, zeros_h1)

  hs2 = pl.pallas_call(
      _comb1_body,
      grid=(GRID,),
      in_specs=[
          pl.BlockSpec((2, RB, H1), lambda i: (0, i, 0)),
          pl.BlockSpec((2, RB, H1), lambda i: (0, i, 0)),
          _deg_spec(),
          pl.BlockSpec((1, D), lambda i: (0, 0)),
          pl.BlockSpec((D, W2P), lambda i: (0, 0)),
      ],
      out_specs=pl.BlockSpec((2, RB, H2), lambda i: (0, i, 0)),
      out_shape=jax.ShapeDtypeStruct((NSC, N, H2), jnp.float32),
  )(p, hs1, degp, b1.reshape(1, D), W2p)

  q = _agg2(hs2, idxi, zeros_h2)

  out = pl.pallas_call(
      _final_body,
      grid=(GRID,),
      in_specs=[
          pl.BlockSpec((2, RB, H2), lambda i: (0, i, 0)),
          pl.BlockSpec((2, RB, H2), lambda i: (0, i, 0)),
          _deg_spec(),
          pl.BlockSpec((1, C), lambda i: (0, 0)),
      ],
      out_specs=pl.BlockSpec((RB, C), lambda i: (i, 0)),
      out_shape=jax.ShapeDtypeStruct((N, C), jnp.float32),
  )(q, hs2, degp, b2.reshape(1, C))

  return out


# speedup vs baseline: 34.7435x; 1.2214x over previous
import functools

import jax
import jax.numpy as jnp
from jax import lax
from jax.experimental import pallas as pl
from jax.experimental.pallas import tpu as pltpu
from jax.experimental.pallas import tpu_sc as plsc

N = 10000
E = 320000
D = 128
C = 40

NSC = 2
NTILE = 16
NW = NSC * NTILE

B = 80
CH = 128
EPT = CH * B
E_PAD = NW * EPT

N_AGG = 10048
SLAB = N_AGG // NTILE

W2P = 48

RB = 1000
GRID = N // RB

_mesh = plsc.VectorSubcoreMesh(core_axis_name="c", subcore_axis_name="s")
_sc_params = pltpu.CompilerParams(use_tc_tiling_on_sc=False)


def _make_agg(W):
  @functools.partial(
      pl.kernel,
      mesh=_mesh,
      compiler_params=_sc_params,
      out_type=jax.ShapeDtypeStruct((NSC, N_AGG, W), jnp.float32),
      scratch_types=[
          pltpu.VMEM((8, 2, B), jnp.int32),
          pltpu.VMEM((4, B, W), jnp.float32),
          pltpu.VMEM_SHARED((N_AGG, W), jnp.float32),
      ]
      + [pltpu.SemaphoreType.DMA] * 16,
  )
  def agg(hs, idxi, zeros_w, out, iring, rows, agg_sh, *sems):
    isems = sems[0:8]
    gsems = sems[8:12]
    ssems = sems[12:16]
    cid = lax.axis_index("c")
    sid = lax.axis_index("s")
    wid = cid * NTILE + sid

    pltpu.sync_copy(
        zeros_w.at[pl.ds(sid * SLAB, SLAB)],
        agg_sh.at[pl.ds(sid * SLAB, SLAB)],
    )
    plsc.subcore_barrier()

    def idx_start(ch, y):
      pltpu.async_copy(idxi.at[wid, ch], iring.at[y], isems[y])

    def idx_wait(ch, y):
      pltpu.make_async_copy(idxi.at[wid, ch], iring.at[y], isems[y]).wait()

    def gat_start(y, x):
      pltpu.async_copy(hs.at[iring.at[y, 0]], rows.at[x], gsems[x])

    def gat_wait(y, x):
      pltpu.make_async_copy(hs.at[iring.at[y, 0]], rows.at[x], gsems[x]).wait()

    def sca_start(y, x):
      pltpu.async_copy(rows.at[x], agg_sh.at[iring.at[y, 1]], ssems[x],
                       add=True)

    def sca_wait(y, x):
      pltpu.make_async_copy(rows.at[x], agg_sh.at[iring.at[y, 1]],
                            ssems[x]).wait()

    for j in range(6):
      idx_start(j, j)
    idx_wait(0, 0)
    gat_start(0, 0)
    idx_wait(1, 1)
    gat_start(1, 1)

    def body(c0):
      for j in range(8):
        ch = c0 + j
        x = j % 4
        y = j % 8
        x2 = (j + 2) % 4
        y2 = (j + 2) % 8
        y6 = (j + 6) % 8

        @pl.when(ch >= 2)
        def _():
          sca_wait(y2, x2)

        @pl.when(ch + 2 < CH)
        def _():
          idx_wait(ch + 2, y2)
          gat_start(y2, x2)

        gat_wait(y, x)
        sca_start(y, x)

        @pl.when(ch + 6 < CH)
        def _():
          idx_start(ch + 6, y6)

    pl.loop(0, CH, step=8)(body)

    sca_wait((CH - 2) % 8, (CH - 2) % 4)
    sca_wait((CH - 1) % 8, (CH - 1) % 4)

    plsc.subcore_barrier()
    pltpu.sync_copy(
        agg_sh.at[pl.ds(sid * SLAB, SLAB)],
        out.at[cid].at[pl.ds(sid * SLAB, SLAB)],
    )

  return agg


_agg1 = _make_agg(D)
_agg2 = _make_agg(W2P)


@functools.partial(
    pl.kernel,
    mesh=_mesh,
    compiler_params=_sc_params,
    out_type=jax.ShapeDtypeStruct((NSC, N_AGG, 16), jnp.float32),
    scratch_types=[
        pltpu.VMEM((8, 2, B), jnp.int32),
        pltpu.VMEM((B, 16), jnp.float32),
        pltpu.VMEM_SHARED((N_AGG, 16), jnp.float32),
    ]
    + [pltpu.SemaphoreType.DMA] * 12,
)
def _deg(idxi, zeros16, out, iring, ones, deg_sh, *sems):
  isems = sems[0:8]
  ssems = sems[8:12]
  cid = lax.axis_index("c")
  sid = lax.axis_index("s")
  wid = cid * NTILE + sid

  for r in range(B):
    ones[r, :] = jnp.full((16,), 1.0, jnp.float32)

  pltpu.sync_copy(
      zeros16.at[pl.ds(sid * SLAB, SLAB)],
      deg_sh.at[pl.ds(sid * SLAB, SLAB)],
  )
  plsc.subcore_barrier()

  def idx_start(ch, y):
    pltpu.async_copy(idxi.at[wid, ch], iring.at[y], isems[y])

  def idx_wait(ch, y):
    pltpu.make_async_copy(idxi.at[wid, ch], iring.at[y], isems[y]).wait()

  def sca_start(y, x):
    pltpu.async_copy(ones, deg_sh.at[iring.at[y, 1]], ssems[x], add=True)

  def sca_wait(y, x):
    pltpu.make_async_copy(ones, deg_sh.at[iring.at[y, 1]], ssems[x]).wait()

  for j in range(4):
    idx_start(j, j)

  def body(c0):
    for j in range(8):
      ch = c0 + j
      x = j % 4
      y = j % 8
      y4 = (j + 4) % 8

      @pl.when(ch >= 4)
      def _():
        sca_wait(y4, x)

      idx_wait(ch, y)
      sca_start(y, x)

      @pl.when(ch + 4 < CH)
      def _():
        idx_start(ch + 4, y4)

  pl.loop(0, CH, step=8)(body)

  for j in range(4):
    sca_wait((CH - 4 + j) % 8, (CH - 4 + j) % 4)

  plsc.subcore_barrier()
  pltpu.sync_copy(
      deg_sh.at[pl.ds(sid * SLAB, SLAB)],
      out.at[cid].at[pl.ds(sid * SLAB, SLAB)],
  )


def _dinv_from(deg_ref):
  degsum = deg_ref[0, :, 0:1] + deg_ref[1, :, 0:1]
  return lax.rsqrt(degsum + 1.0)


def _mm1_body(x_ref, w_ref, deg_ref, o_ref):
  dinv = _dinv_from(deg_ref)
  o_ref[...] = dinv * jnp.dot(
      x_ref[...], w_ref[...], preferred_element_type=jnp.float32
  )


def _comb1_body(p_ref, hs1_ref, deg_ref, b1_ref, w2_ref, o_ref):
  dinv = _dinv_from(deg_ref)
  h1 = jnp.maximum(
      dinv * (p_ref[0] + p_ref[1] + hs1_ref[...]) + b1_ref[...], 0.0
  )
  o_ref[...] = dinv * jnp.dot(
      h1, w2_ref[...], preferred_element_type=jnp.float32
  )


def _final_body(q_ref, hs2_ref, deg_ref, b2_ref, o_ref):
  dinv = _dinv_from(deg_ref)
  z = dinv * (q_ref[0] + q_ref[1] + hs2_ref[...])
  z40 = z[:, :C] + b2_ref[...]
  m = jnp.max(z40, axis=1, keepdims=True)
  e = z40 - m
  o_ref[...] = e - jnp.log(jnp.sum(jnp.exp(e), axis=1, keepdims=True))


def _deg_spec():
  return pl.BlockSpec((2, RB, 16), lambda i: (0, i, 0))


def kernel(x, edge_index, W1, b1, W2, b2):
  row = edge_index[0]
  col = edge_index[1]

  npad = E_PAD - E
  pad_pos = jnp.arange(npad, dtype=jnp.int32)
  row_pad = pad_pos % jnp.int32(9973)
  col_pad = jnp.int32(N) + pad_pos % jnp.int32(N_AGG - N)
  rowf = jnp.concatenate([row, row_pad]).reshape(NW, CH, B)
  colf = jnp.concatenate([col, col_pad]).reshape(NW, CH, B)
  idxi = jnp.stack([rowf, colf], axis=2)

  zeros16 = jnp.zeros((N_AGG, 16), jnp.float32)
  zeros_d = jnp.zeros((N_AGG, D), jnp.float32)
  zeros_w2 = jnp.zeros((N_AGG, W2P), jnp.float32)
  W2p = jnp.pad(W2, ((0, 0), (0, W2P - C)))

  degp = _deg(idxi, zeros16)

  hs1 = pl.pallas_call(
      _mm1_body,
      grid=(GRID,),
      in_specs=[
          pl.BlockSpec((RB, D), lambda i: (i, 0)),
          pl.BlockSpec((D, D), lambda i: (0, 0)),
          _deg_spec(),
      ],
      out_specs=pl.BlockSpec((RB, D), lambda i: (i, 0)),
      out_shape=jax.ShapeDtypeStruct((N, D), jnp.float32),
  )(x, W1, degp)

  p = _agg1(hs1, idxi, zeros_d)

  hs2 = pl.pallas_call(
      _comb1_body,
      grid=(GRID,),
      in_specs=[
          pl.BlockSpec((2, RB, D), lambda i: (0, i, 0)),
          pl.BlockSpec((RB, D), lambda i: (i, 0)),
          _deg_spec(),
          pl.BlockSpec((1, D), lambda i: (0, 0)),
          pl.BlockSpec((D, W2P), lambda i: (0, 0)),
      ],
      out_specs=pl.BlockSpec((RB, W2P), lambda i: (i, 0)),
      out_shape=jax.ShapeDtypeStruct((N, W2P), jnp.float32),
  )(p, hs1, degp, b1.reshape(1, D), W2p)

  q = _agg2(hs2, idxi, zeros_w2)

  out = pl.pallas_call(
      _final_body,
      grid=(GRID,),
      in_specs=[
          pl.BlockSpec((2, RB, W2P), lambda i: (0, i, 0)),
          pl.BlockSpec((RB, W2P), lambda i: (i, 0)),
          _deg_spec(),
          pl.BlockSpec((1, C), lambda i: (0, 0)),
      ],
      out_specs=pl.BlockSpec((RB, C), lambda i: (i, 0)),
      out_shape=jax.ShapeDtypeStruct((N, C), jnp.float32),
  )(q, hs2, degp, b2.reshape(1, C))

  return out


# speedup vs baseline: 39.2583x; 1.1299x over previous
import functools

import jax
import jax.numpy as jnp
from jax import lax
from jax.experimental import pallas as pl
from jax.experimental.pallas import tpu as pltpu
from jax.experimental.pallas import tpu_sc as plsc

N = 10000
E = 320000
D = 128
C = 40

NSC = 2
NTILE = 16
NW = NSC * NTILE

B = 80
CH = 125
CH_UP = 128
EPT = CH * B

N_AGG = N
SLAB = N_AGG // NTILE

W2P = 48

RB = 1000
GRID = N // RB

_mesh = plsc.VectorSubcoreMesh(core_axis_name="c", subcore_axis_name="s")
_sc_params = pltpu.CompilerParams(use_tc_tiling_on_sc=False)


def _make_agg(W):
  @functools.partial(
      pl.kernel,
      mesh=_mesh,
      compiler_params=_sc_params,
      out_type=jax.ShapeDtypeStruct((NSC, N_AGG, W), jnp.float32),
      scratch_types=[
          pltpu.VMEM((8, 2, B), jnp.int32),
          pltpu.VMEM((4, B, W), jnp.float32),
          pltpu.VMEM_SHARED((N_AGG, W), jnp.float32),
      ]
      + [pltpu.SemaphoreType.DMA] * 16,
  )
  def agg(hs, eidx, zeros_w, out, iring, rows, agg_sh, *sems):
    isems = sems[0:8]
    gsems = sems[8:12]
    ssems = sems[12:16]
    cid = lax.axis_index("c")
    sid = lax.axis_index("s")
    wid = cid * NTILE + sid

    pltpu.sync_copy(
        zeros_w.at[pl.ds(sid * SLAB, SLAB)],
        agg_sh.at[pl.ds(sid * SLAB, SLAB)],
    )
    plsc.subcore_barrier()

    def idx_start(ch, y):
      base = wid * EPT + ch * B
      pltpu.async_copy(eidx.at[0, pl.ds(base, B)], iring.at[y, 0], isems[y])
      pltpu.async_copy(eidx.at[1, pl.ds(base, B)], iring.at[y, 1], isems[y])

    def idx_wait(ch, y):
      base = wid * EPT + ch * B
      pltpu.make_async_copy(
          eidx.at[0, pl.ds(base, B)], iring.at[y, 0], isems[y]
      ).wait()
      pltpu.make_async_copy(
          eidx.at[1, pl.ds(base, B)], iring.at[y, 1], isems[y]
      ).wait()

    def gat_start(y, x):
      pltpu.async_copy(hs.at[iring.at[y, 0]], rows.at[x], gsems[x])

    def gat_wait(y, x):
      pltpu.make_async_copy(hs.at[iring.at[y, 0]], rows.at[x], gsems[x]).wait()

    def sca_start(y, x):
      pltpu.async_copy(rows.at[x], agg_sh.at[iring.at[y, 1]], ssems[x],
                       add=True)

    def sca_wait(y, x):
      pltpu.make_async_copy(rows.at[x], agg_sh.at[iring.at[y, 1]],
                            ssems[x]).wait()

    for j in range(6):
      idx_start(j, j)
    idx_wait(0, 0)
    gat_start(0, 0)
    idx_wait(1, 1)
    gat_start(1, 1)

    def body(c0):
      for j in range(8):
        ch = c0 + j
        x = j % 4
        y = j % 8
        x2 = (j + 2) % 4
        y2 = (j + 2) % 8
        y6 = (j + 6) % 8

        @pl.when((ch >= 2) & (ch < CH + 2))
        def _():
          sca_wait(y2, x2)

        @pl.when(ch + 2 < CH)
        def _():
          idx_wait(ch + 2, y2)
          gat_start(y2, x2)

        @pl.when(ch < CH)
        def _():
          gat_wait(y, x)
          sca_start(y, x)

        @pl.when(ch + 6 < CH)
        def _():
          idx_start(ch + 6, y6)

    pl.loop(0, CH_UP, step=8)(body)

    plsc.subcore_barrier()
    pltpu.sync_copy(
        agg_sh.at[pl.ds(sid * SLAB, SLAB)],
        out.at[cid].at[pl.ds(sid * SLAB, SLAB)],
    )

  return agg


_agg1 = _make_agg(D)
_agg2 = _make_agg(W2P)


@functools.partial(
    pl.kernel,
    mesh=_mesh,
    compiler_params=_sc_params,
    out_type=jax.ShapeDtypeStruct((NSC, N_AGG, 16), jnp.float32),
    scratch_types=[
        pltpu.VMEM((8, 2, B), jnp.int32),
        pltpu.VMEM((B, 16), jnp.float32),
        pltpu.VMEM_SHARED((N_AGG, 16), jnp.float32),
    ]
    + [pltpu.SemaphoreType.DMA] * 12,
)
def _deg(eidx, zeros16, out, iring, ones, deg_sh, *sems):
  isems = sems[0:8]
  ssems = sems[8:12]
  cid = lax.axis_index("c")
  sid = lax.axis_index("s")
  wid = cid * NTILE + sid

  for r in range(B):
    ones[r, :] = jnp.full((16,), 1.0, jnp.float32)

  pltpu.sync_copy(
      zeros16.at[pl.ds(sid * SLAB, SLAB)],
      deg_sh.at[pl.ds(sid * SLAB, SLAB)],
  )
  plsc.subcore_barrier()

  def idx_start(ch, y):
    base = wid * EPT + ch * B
    pltpu.async_copy(eidx.at[1, pl.ds(base, B)], iring.at[y, 1], isems[y])

  def idx_wait(ch, y):
    base = wid * EPT + ch * B
    pltpu.make_async_copy(
        eidx.at[1, pl.ds(base, B)], iring.at[y, 1], isems[y]
    ).wait()

  def sca_start(y, x):
    pltpu.async_copy(ones, deg_sh.at[iring.at[y, 1]], ssems[x], add=True)

  def sca_wait(y, x):
    pltpu.make_async_copy(ones, deg_sh.at[iring.at[y, 1]], ssems[x]).wait()

  for j in range(4):
    idx_start(j, j)

  def body(c0):
    for j in range(8):
      ch = c0 + j
      x = j % 4
      y = j % 8
      y4 = (j + 4) % 8

      @pl.when((ch >= 4) & (ch < CH + 4))
      def _():
        sca_wait(y4, x)

      @pl.when(ch < CH)
      def _():
        idx_wait(ch, y)
        sca_start(y, x)

      @pl.when(ch + 4 < CH)
      def _():
        idx_start(ch + 4, y4)

  pl.loop(0, CH_UP, step=8)(body)

  sca_wait(4, 0)

  plsc.subcore_barrier()
  pltpu.sync_copy(
      deg_sh.at[pl.ds(sid * SLAB, SLAB)],
      out.at[cid].at[pl.ds(sid * SLAB, SLAB)],
  )


def _dinv_from(deg_ref):
  degsum = deg_ref[0, :, 0:1] + deg_ref[1, :, 0:1]
  return lax.rsqrt(degsum + 1.0)


def _mm1_body(x_ref, w_ref, deg_ref, o_ref):
  dinv = _dinv_from(deg_ref)
  o_ref[...] = dinv * jnp.dot(
      x_ref[...], w_ref[...], preferred_element_type=jnp.float32
  )


def _comb1_body(p_ref, hs1_ref, deg_ref, b1_ref, w2_ref, o_ref):
  dinv = _dinv_from(deg_ref)
  h1 = jnp.maximum(
      dinv * (p_ref[0] + p_ref[1] + hs1_ref[...]) + b1_ref[...], 0.0
  )
  o_ref[...] = dinv * jnp.dot(
      h1, w2_ref[...], preferred_element_type=jnp.float32
  )


def _final_body(q_ref, hs2_ref, deg_ref, b2_ref, o_ref):
  dinv = _dinv_from(deg_ref)
  z = dinv * (q_ref[0] + q_ref[1] + hs2_ref[...])
  z40 = z[:, :C] + b2_ref[...]
  m = jnp.max(z40, axis=1, keepdims=True)
  e = z40 - m
  o_ref[...] = e - jnp.log(jnp.sum(jnp.exp(e), axis=1, keepdims=True))


def _deg_spec():
  return pl.BlockSpec((2, RB, 16), lambda i: (0, i, 0))


def kernel(x, edge_index, W1, b1, W2, b2):
  zeros16 = jnp.zeros((N_AGG, 16), jnp.float32)
  zeros_d = jnp.zeros((N_AGG, D), jnp.float32)
  zeros_w2 = jnp.zeros((N_AGG, W2P), jnp.float32)
  W2p = jnp.pad(W2, ((0, 0), (0, W2P - C)))

  degp = _deg(edge_index, zeros16)

  hs1 = pl.pallas_call(
      _mm1_body,
      grid=(GRID,),
      in_specs=[
          pl.BlockSpec((RB, D), lambda i: (i, 0)),
          pl.BlockSpec((D, D), lambda i: (0, 0)),
          _deg_spec(),
      ],
      out_specs=pl.BlockSpec((RB, D), lambda i: (i, 0)),
      out_shape=jax.ShapeDtypeStruct((N, D), jnp.float32),
  )(x, W1, degp)

  p = _agg1(hs1, edge_index, zeros_d)

  hs2 = pl.pallas_call(
      _comb1_body,
      grid=(GRID,),
      in_specs=[
          pl.BlockSpec((2, RB, D), lambda i: (0, i, 0)),
          pl.BlockSpec((RB, D), lambda i: (i, 0)),
          _deg_spec(),
          pl.BlockSpec((1, D), lambda i: (0, 0)),
          pl.BlockSpec((D, W2P), lambda i: (0, 0)),
      ],
      out_specs=pl.BlockSpec((RB, W2P), lambda i: (i, 0)),
      out_shape=jax.ShapeDtypeStruct((N, W2P), jnp.float32),
  )(p, hs1, degp, b1.reshape(1, D), W2p)

  q = _agg2(hs2, edge_index, zeros_w2)

  out = pl.pallas_call(
      _final_body,
      grid=(GRID,),
      in_specs=[
          pl.BlockSpec((2, RB, W2P), lambda i: (0, i, 0)),
          pl.BlockSpec((RB, W2P), lambda i: (i, 0)),
          _deg_spec(),
          pl.BlockSpec((1, C), lambda i: (0, 0)),
      ],
      out_specs=pl.BlockSpec((RB, C), lambda i: (i, 0)),
      out_shape=jax.ShapeDtypeStruct((N, C), jnp.float32),
  )(q, hs2, degp, b2.reshape(1, C))

  return out


# speedup vs baseline: 39.3517x; 1.0024x over previous
import functools

import jax
import jax.numpy as jnp
from jax import lax
from jax.experimental import pallas as pl
from jax.experimental.pallas import tpu as pltpu
from jax.experimental.pallas import tpu_sc as plsc

N = 10000
E = 320000
D = 128
C = 40

NSC = 2
NTILE = 16
NW = NSC * NTILE

B = 80
CH = 125
CH_UP = 128
EPT = CH * B

N_AGG = 10112
SLAB = N_AGG // NTILE

W2P = 48

RB = 1000
GRID = N // RB

_mesh = plsc.VectorSubcoreMesh(core_axis_name="c", subcore_axis_name="s")
_sc_params = pltpu.CompilerParams(use_tc_tiling_on_sc=False)


def _make_agg(W, tc_tiling):
  @functools.partial(
      pl.kernel,
      mesh=_mesh,
      compiler_params=None if tc_tiling else _sc_params,
      out_type=jax.ShapeDtypeStruct((NSC, N_AGG, W), jnp.float32),
      scratch_types=[
          pltpu.VMEM((8, 2, B), jnp.int32),
          pltpu.VMEM((4, B, W), jnp.float32),
          pltpu.VMEM_SHARED((N_AGG, W), jnp.float32),
      ]
      + [pltpu.SemaphoreType.DMA] * 16,
  )
  def agg(hs, eidx, zeros_w, out, iring, rows, agg_sh, *sems):
    isems = sems[0:8]
    gsems = sems[8:12]
    ssems = sems[12:16]
    cid = lax.axis_index("c")
    sid = lax.axis_index("s")
    wid = cid * NTILE + sid

    pltpu.sync_copy(
        zeros_w.at[pl.ds(sid * SLAB, SLAB)],
        agg_sh.at[pl.ds(sid * SLAB, SLAB)],
    )
    plsc.subcore_barrier()

    def idx_start(ch, y):
      base = wid * EPT + ch * B
      pltpu.async_copy(eidx.at[pl.ds(base, B)], iring.at[y, 0], isems[y])
      pltpu.async_copy(eidx.at[pl.ds(E + base, B)], iring.at[y, 1], isems[y])

    def idx_wait(ch, y):
      base = wid * EPT + ch * B
      pltpu.make_async_copy(
          eidx.at[pl.ds(base, B)], iring.at[y, 0], isems[y]
      ).wait()
      pltpu.make_async_copy(
          eidx.at[pl.ds(E + base, B)], iring.at[y, 1], isems[y]
      ).wait()

    def gat_start(y, x):
      pltpu.async_copy(hs.at[iring.at[y, 0]], rows.at[x], gsems[x])

    def gat_wait(y, x):
      pltpu.make_async_copy(hs.at[iring.at[y, 0]], rows.at[x], gsems[x]).wait()

    def sca_start(y, x):
      pltpu.async_copy(rows.at[x], agg_sh.at[iring.at[y, 1]], ssems[x],
                       add=True)

    def sca_wait(y, x):
      pltpu.make_async_copy(rows.at[x], agg_sh.at[iring.at[y, 1]],
                            ssems[x]).wait()

    for j in range(6):
      idx_start(j, j)
    idx_wait(0, 0)
    gat_start(0, 0)
    idx_wait(1, 1)
    gat_start(1, 1)

    def body(c0):
      for j in range(8):
        ch = c0 + j
        x = j % 4
        y = j % 8
        x2 = (j + 2) % 4
        y2 = (j + 2) % 8
        y6 = (j + 6) % 8

        @pl.when((ch >= 2) & (ch < CH + 2))
        def _():
          sca_wait(y2, x2)

        @pl.when(ch + 2 < CH)
        def _():
          idx_wait(ch + 2, y2)
          gat_start(y2, x2)

        @pl.when(ch < CH)
        def _():
          gat_wait(y, x)
          sca_start(y, x)

        @pl.when(ch + 6 < CH)
        def _():
          idx_start(ch + 6, y6)

    pl.loop(0, CH_UP, step=8)(body)

    plsc.subcore_barrier()
    pltpu.sync_copy(
        agg_sh.at[pl.ds(sid * SLAB, SLAB)],
        out.at[cid].at[pl.ds(sid * SLAB, SLAB)],
    )

  return agg


_agg1 = _make_agg(D, tc_tiling=True)
_agg2 = _make_agg(W2P, tc_tiling=False)


@functools.partial(
    pl.kernel,
    mesh=_mesh,
    compiler_params=_sc_params,
    out_type=jax.ShapeDtypeStruct((NSC, N_AGG, 16), jnp.float32),
    scratch_types=[
        pltpu.VMEM((8, 2, B), jnp.int32),
        pltpu.VMEM((B, 16), jnp.float32),
        pltpu.VMEM_SHARED((N_AGG, 16), jnp.float32),
    ]
    + [pltpu.SemaphoreType.DMA] * 12,
)
def _deg(eidx, zeros16, out, iring, ones, deg_sh, *sems):
  isems = sems[0:8]
  ssems = sems[8:12]
  cid = lax.axis_index("c")
  sid = lax.axis_index("s")
  wid = cid * NTILE + sid

  for r in range(B):
    ones[r, :] = jnp.full((16,), 1.0, jnp.float32)

  pltpu.sync_copy(
      zeros16.at[pl.ds(sid * SLAB, SLAB)],
      deg_sh.at[pl.ds(sid * SLAB, SLAB)],
  )
  plsc.subcore_barrier()

  def idx_start(ch, y):
    base = E + wid * EPT + ch * B
    pltpu.async_copy(eidx.at[pl.ds(base, B)], iring.at[y, 1], isems[y])

  def idx_wait(ch, y):
    base = E + wid * EPT + ch * B
    pltpu.make_async_copy(
        eidx.at[pl.ds(base, B)], iring.at[y, 1], isems[y]
    ).wait()

  def sca_start(y, x):
    pltpu.async_copy(ones, deg_sh.at[iring.at[y, 1]], ssems[x], add=True)

  def sca_wait(y, x):
    pltpu.make_async_copy(ones, deg_sh.at[iring.at[y, 1]], ssems[x]).wait()

  for j in range(4):
    idx_start(j, j)

  def body(c0):
    for j in range(8):
      ch = c0 + j
      x = j % 4
      y = j % 8
      y4 = (j + 4) % 8

      @pl.when((ch >= 4) & (ch < CH + 4))
      def _():
        sca_wait(y4, x)

      @pl.when(ch < CH)
      def _():
        idx_wait(ch, y)
        sca_start(y, x)

      @pl.when(ch + 4 < CH)
      def _():
        idx_start(ch + 4, y4)

  pl.loop(0, CH_UP, step=8)(body)

  sca_wait(4, 0)

  plsc.subcore_barrier()
  pltpu.sync_copy(
      deg_sh.at[pl.ds(sid * SLAB, SLAB)],
      out.at[cid].at[pl.ds(sid * SLAB, SLAB)],
  )


def _dinv_from(deg_ref):
  degsum = deg_ref[0, :, 0:1] + deg_ref[1, :, 0:1]
  return lax.rsqrt(degsum + 1.0)


def _mm1_body(x_ref, w_ref, deg_ref, o_ref):
  dinv = _dinv_from(deg_ref)
  o_ref[...] = dinv * jnp.dot(
      x_ref[...], w_ref[...], preferred_element_type=jnp.float32
  )


def _comb1_body(p_ref, hs1_ref, deg_ref, b1_ref, w2_ref, o_ref):
  dinv = _dinv_from(deg_ref)
  h1 = jnp.maximum(
      dinv * (p_ref[0] + p_ref[1] + hs1_ref[...]) + b1_ref[...], 0.0
  )
  o_ref[...] = dinv * jnp.dot(
      h1, w2_ref[...], preferred_element_type=jnp.float32
  )


def _final_body(q_ref, hs2_ref, deg_ref, b2_ref, o_ref):
  dinv = _dinv_from(deg_ref)
  z = dinv * (q_ref[0] + q_ref[1] + hs2_ref[...])
  z40 = z[:, :C] + b2_ref[...]
  m = jnp.max(z40, axis=1, keepdims=True)
  e = z40 - m
  o_ref[...] = e - jnp.log(jnp.sum(jnp.exp(e), axis=1, keepdims=True))


def _deg_spec():
  return pl.BlockSpec((2, RB, 16), lambda i: (0, i, 0))


def kernel(x, edge_index, W1, b1, W2, b2):
  zeros16 = jnp.zeros((N_AGG, 16), jnp.float32)
  zeros_d = jnp.zeros((N_AGG, D), jnp.float32)
  zeros_w2 = jnp.zeros((N_AGG, W2P), jnp.float32)
  W2p = jnp.pad(W2, ((0, 0), (0, W2P - C)))

  eidx1 = edge_index.reshape(2 * E)
  degp = _deg(eidx1, zeros16)

  hs1 = pl.pallas_call(
      _mm1_body,
      grid=(GRID,),
      in_specs=[
          pl.BlockSpec((RB, D), lambda i: (i, 0)),
          pl.BlockSpec((D, D), lambda i: (0, 0)),
          _deg_spec(),
      ],
      out_specs=pl.BlockSpec((RB, D), lambda i: (i, 0)),
      out_shape=jax.ShapeDtypeStruct((N, D), jnp.float32),
  )(x, W1, degp)

  p = _agg1(hs1, eidx1, zeros_d)

  hs2 = pl.pallas_call(
      _comb1_body,
      grid=(GRID,),
      in_specs=[
          pl.BlockSpec((2, RB, D), lambda i: (0, i, 0)),
          pl.BlockSpec((RB, D), lambda i: (i, 0)),
          _deg_spec(),
          pl.BlockSpec((1, D), lambda i: (0, 0)),
          pl.BlockSpec((D, W2P), lambda i: (0, 0)),
      ],
      out_specs=pl.BlockSpec((RB, W2P), lambda i: (i, 0)),
      out_shape=jax.ShapeDtypeStruct((N, W2P), jnp.float32),
  )(p, hs1, degp, b1.reshape(1, D), W2p)

  q = _agg2(hs2, eidx1, zeros_w2)

  out = pl.pallas_call(
      _final_body,
      grid=(GRID,),
      in_specs=[
          pl.BlockSpec((2, RB, W2P), lambda i: (0, i, 0)),
          pl.BlockSpec((RB, W2P), lambda i: (i, 0)),
          _deg_spec(),
          pl.BlockSpec((1, C), lambda i: (0, 0)),
      ],
      out_specs=pl.BlockSpec((RB, C), lambda i: (i, 0)),
      out_shape=jax.ShapeDtypeStruct((N, C), jnp.float32),
  )(q, hs2, degp, b2.reshape(1, C))

  return out


# speedup vs baseline: 40.0584x; 1.0180x over previous
import functools

import jax
import jax.numpy as jnp
from jax import lax
from jax.experimental import pallas as pl
from jax.experimental.pallas import tpu as pltpu
from jax.experimental.pallas import tpu_sc as plsc

N = 10000
E = 320000
D = 128
C = 40

NSC = 2
NTILE = 16
NW = NSC * NTILE

B = 80
CH = 125
CH_UP = 128
EPT = CH * B

N_AGG = 10112
SLAB = N_AGG // NTILE

W2P = 48

RB = 1000
GRID = N // RB

_mesh = plsc.VectorSubcoreMesh(core_axis_name="c", subcore_axis_name="s")
_sc_params = pltpu.CompilerParams(use_tc_tiling_on_sc=False)


def _make_agg(W, tc_tiling):
  @functools.partial(
      pl.kernel,
      mesh=_mesh,
      compiler_params=None if tc_tiling else _sc_params,
      out_type=jax.ShapeDtypeStruct((NSC, N_AGG, W), jnp.float32),
      scratch_types=[
          pltpu.VMEM((8, 2, B), jnp.int32),
          pltpu.VMEM((4, B, W), jnp.float32),
          pltpu.VMEM_SHARED((N_AGG, W), jnp.float32),
      ]
      + [pltpu.SemaphoreType.DMA] * 16,
  )
  def agg(hs, eidx, zeros_w, out, iring, rows, agg_sh, *sems):
    isems = sems[0:8]
    gsems = sems[8:12]
    ssems = sems[12:16]
    cid = lax.axis_index("c")
    sid = lax.axis_index("s")
    wid = cid * NTILE + sid

    pltpu.sync_copy(
        zeros_w.at[pl.ds(sid * SLAB, SLAB)],
        agg_sh.at[pl.ds(sid * SLAB, SLAB)],
    )
    plsc.subcore_barrier()

    def idx_start(ch, y):
      base = wid * EPT + ch * B
      pltpu.async_copy(eidx.at[pl.ds(base, B)], iring.at[y, 0], isems[y])
      pltpu.async_copy(eidx.at[pl.ds(E + base, B)], iring.at[y, 1], isems[y])

    def idx_wait(ch, y):
      base = wid * EPT + ch * B
      pltpu.make_async_copy(
          eidx.at[pl.ds(base, B)], iring.at[y, 0], isems[y]
      ).wait()
      pltpu.make_async_copy(
          eidx.at[pl.ds(E + base, B)], iring.at[y, 1], isems[y]
      ).wait()

    def gat_start(y, x):
      pltpu.async_copy(hs.at[iring.at[y, 0]], rows.at[x], gsems[x])

    def gat_wait(y, x):
      pltpu.make_async_copy(hs.at[iring.at[y, 0]], rows.at[x], gsems[x]).wait()

    def sca_start(y, x):
      pltpu.async_copy(rows.at[x], agg_sh.at[iring.at[y, 1]], ssems[x],
                       add=True)

    def sca_wait(y, x):
      pltpu.make_async_copy(rows.at[x], agg_sh.at[iring.at[y, 1]],
                            ssems[x]).wait()

    for j in range(6):
      idx_start(j, j)
    idx_wait(0, 0)
    gat_start(0, 0)
    idx_wait(1, 1)
    gat_start(1, 1)

    def body(c0):
      for j in range(8):
        ch = c0 + j
        x = j % 4
        y = j % 8
        x2 = (j + 2) % 4
        y2 = (j + 2) % 8
        y6 = (j + 6) % 8

        @pl.when((ch >= 2) & (ch < CH + 2))
        def _():
          sca_wait(y2, x2)

        @pl.when(ch + 2 < CH)
        def _():
          idx_wait(ch + 2, y2)
          gat_start(y2, x2)

        @pl.when(ch < CH)
        def _():
          gat_wait(y, x)
          sca_start(y, x)

        @pl.when(ch + 6 < CH)
        def _():
          idx_start(ch + 6, y6)

    pl.loop(0, CH_UP, step=8)(body)

    plsc.subcore_barrier()
    pltpu.sync_copy(
        agg_sh.at[pl.ds(sid * SLAB, SLAB)],
        out.at[cid].at[pl.ds(sid * SLAB, SLAB)],
    )

  return agg


_agg1 = _make_agg(D, tc_tiling=True)
_agg2 = _make_agg(W2P, tc_tiling=False)


@functools.partial(
    pl.kernel,
    mesh=_mesh,
    compiler_params=_sc_params,
    out_type=jax.ShapeDtypeStruct((NSC, N_AGG, 16), jnp.float32),
    scratch_types=[
        pltpu.VMEM((8, 2, B), jnp.int32),
        pltpu.VMEM((B, 16), jnp.float32),
        pltpu.VMEM_SHARED((N_AGG, 16), jnp.float32),
    ]
    + [pltpu.SemaphoreType.DMA] * 12,
)
def _deg(eidx, zeros16, out, iring, ones, deg_sh, *sems):
  isems = sems[0:8]
  ssems = sems[8:12]
  cid = lax.axis_index("c")
  sid = lax.axis_index("s")
  wid = cid * NTILE + sid

  for r in range(B):
    ones[r, :] = jnp.full((16,), 1.0, jnp.float32)

  pltpu.sync_copy(
      zeros16.at[pl.ds(sid * SLAB, SLAB)],
      deg_sh.at[pl.ds(sid * SLAB, SLAB)],
  )
  plsc.subcore_barrier()

  def idx_start(ch, y):
    base = E + wid * EPT + ch * B
    pltpu.async_copy(eidx.at[pl.ds(base, B)], iring.at[y, 1], isems[y])

  def idx_wait(ch, y):
    base = E + wid * EPT + ch * B
    pltpu.make_async_copy(
        eidx.at[pl.ds(base, B)], iring.at[y, 1], isems[y]
    ).wait()

  def sca_start(y, x):
    pltpu.async_copy(ones, deg_sh.at[iring.at[y, 1]], ssems[x], add=True)

  def sca_wait(y, x):
    pltpu.make_async_copy(ones, deg_sh.at[iring.at[y, 1]], ssems[x]).wait()

  for j in range(4):
    idx_start(j, j)

  def body(c0):
    for j in range(8):
      ch = c0 + j
      x = j % 4
      y = j % 8
      y4 = (j + 4) % 8

      @pl.when((ch >= 4) & (ch < CH + 4))
      def _():
        sca_wait(y4, x)

      @pl.when(ch < CH)
      def _():
        idx_wait(ch, y)
        sca_start(y, x)

      @pl.when(ch + 4 < CH)
      def _():
        idx_start(ch + 4, y4)

  pl.loop(0, CH_UP, step=8)(body)

  sca_wait(4, 0)

  plsc.subcore_barrier()
  pltpu.sync_copy(
      deg_sh.at[pl.ds(sid * SLAB, SLAB)],
      out.at[cid].at[pl.ds(sid * SLAB, SLAB)],
  )


def _dinv_from(deg_ref):
  degsum = deg_ref[0, :, 0:1] + deg_ref[1, :, 0:1]
  return lax.rsqrt(degsum + 1.0)


def _mm1_body(x_ref, w_ref, deg_ref, o_ref):
  dinv = _dinv_from(deg_ref)
  o_ref[...] = dinv * jnp.dot(
      x_ref[...], w_ref[...], preferred_element_type=jnp.float32
  )


def _comb1_body(p_ref, hs1_ref, deg_ref, b1_ref, w2_ref, o_ref):
  dinv = _dinv_from(deg_ref)
  h1 = jnp.maximum(
      dinv * (p_ref[0] + p_ref[1] + hs1_ref[...]) + b1_ref[...], 0.0
  )
  o_ref[...] = dinv * jnp.dot(
      h1, w2_ref[...], preferred_element_type=jnp.float32
  )


def _final_body(q_ref, hs2_ref, deg_ref, b2_ref, o_ref):
  degsum = deg_ref[0, :N, 0:1] + deg_ref[1, :N, 0:1]
  dinv = lax.rsqrt(degsum + 1.0)
  z = dinv * (q_ref[0, :N, :] + q_ref[1, :N, :] + hs2_ref[...])
  z40 = z[:, :C] + b2_ref[...]
  m = jnp.max(z40, axis=1, keepdims=True)
  e = z40 - m
  ls = e - jnp.log(jnp.sum(jnp.exp(e), axis=1, keepdims=True))
  o_ref[...] = ls.T


def _deg_spec():
  return pl.BlockSpec((2, RB, 16), lambda i: (0, i, 0))


def kernel(x, edge_index, W1, b1, W2, b2):
  zeros16 = jnp.zeros((N_AGG, 16), jnp.float32)
  zeros_d = jnp.zeros((N_AGG, D), jnp.float32)
  zeros_w2 = jnp.zeros((N_AGG, W2P), jnp.float32)
  W2p = jnp.pad(W2, ((0, 0), (0, W2P - C)))

  eidx1 = edge_index.reshape(2 * E)
  degp = _deg(eidx1, zeros16)

  hs1 = pl.pallas_call(
      _mm1_body,
      grid=(GRID,),
      in_specs=[
          pl.BlockSpec((RB, D), lambda i: (i, 0)),
          pl.BlockSpec((D, D), lambda i: (0, 0)),
          _deg_spec(),
      ],
      out_specs=pl.BlockSpec((RB, D), lambda i: (i, 0)),
      out_shape=jax.ShapeDtypeStruct((N, D), jnp.float32),
  )(x, W1, degp)

  p = _agg1(hs1, eidx1, zeros_d)

  hs2 = pl.pallas_call(
      _comb1_body,
      grid=(GRID,),
      in_specs=[
          pl.BlockSpec((2, RB, D), lambda i: (0, i, 0)),
          pl.BlockSpec((RB, D), lambda i: (i, 0)),
          _deg_spec(),
          pl.BlockSpec((1, D), lambda i: (0, 0)),
          pl.BlockSpec((D, W2P), lambda i: (0, 0)),
      ],
      out_specs=pl.BlockSpec((RB, W2P), lambda i: (i, 0)),
      out_shape=jax.ShapeDtypeStruct((N, W2P), jnp.float32),
  )(p, hs1, degp, b1.reshape(1, D), W2p)

  q = _agg2(hs2, eidx1, zeros_w2)

  out_t = pl.pallas_call(
      _final_body,
      out_shape=jax.ShapeDtypeStruct((C, N), jnp.float32),
  )(q, hs2, degp, b2.reshape(1, C))

  return out_t.T
